# baseline (speedup 1.0000x reference)
import jax
import jax.numpy as jnp
import numpy as np
from jax.experimental import pallas as pl
from jax.experimental.pallas import tpu as pltpu

_L = 2048
_D = 1024
_HQ = 16
_HKV = 4
_HD = 64
_NREP = _HQ // _HKV
_BLK = 128
_NB = _L // _BLK
_TOPK = 8
_CH = 512
_NCH = _L // _CH
_BPC = _CH // _BLK
_SCALE = 1.0 / np.sqrt(_HD)
_NEG = -1e9


def _rope_halves(z, c, s):
    a = z[:, : _HD // 2]
    b = z[:, _HD // 2:]
    return jnp.concatenate([a * c - b * s, a * s + b * c], axis=1)


def _attn_body(x_ref, cos_ref, sin_ref, wq_ref, wk_ref, wv_ref, wo_ref,
               y_ref, kc_ref, vc_ref, sc_ref, acc_ref):
    qb = pl.program_id(0)
    h = pl.program_id(1)

    xb = x_ref[...]
    c = cos_ref[...]
    s = sin_ref[...]

    @pl.when(jnp.logical_and(qb == 0, h == 0))
    def _zero_v():
        vc_ref[...] = jnp.zeros((_HKV, _L, 2 * _HD), jnp.float32)

    @pl.when(h == 0)
    def _kv():
        extra = (jax.lax.broadcasted_iota(jnp.int32, (_BLK, _HD), 1) == 0
                 ).astype(jnp.float32)
        for g in range(_HKV):
            kg = jax.lax.dot_general(xb, wk_ref[g], (((1,), (0,)), ((), ())),
                                     preferred_element_type=jnp.float32)
            kc_ref[g, pl.ds(qb * _BLK, _BLK), :] = _rope_halves(kg, c, s)
            vg = jax.lax.dot_general(xb, wv_ref[g], (((1,), (0,)), ((), ())),
                                     preferred_element_type=jnp.float32)
            vc_ref[g, pl.ds(qb * _BLK, _BLK), :] = jnp.concatenate(
                [vg, extra], axis=1)

    qh = jax.lax.dot_general(xb, wq_ref[h], (((1,), (0,)), ((), ())),
                             preferred_element_type=jnp.float32)
    qr = _rope_halves(qh, c, s)

    g = h // _NREP
    row = qb * _BLK + jax.lax.broadcasted_iota(jnp.int32, (_BLK, _CH), 0)

    for ci in range(_NCH):
        @pl.when(qb >= _BPC * ci)
        def _qk(ci=ci):
            kk = kc_ref[g, pl.ds(ci * _CH, _CH), :]
            sch = jax.lax.dot_general(qr, kk, (((1,), (1,)), ((), ())),
                                      preferred_element_type=jnp.float32)
            col = ci * _CH + jax.lax.broadcasted_iota(
                jnp.int32, (_BLK, _CH), 1)
            sc_ref[:, ci * _CH:(ci + 1) * _CH] = jnp.where(
                col <= row, sch * _SCALE, _NEG)

    bms = [jnp.where(j <= qb,
                     jnp.max(sc_ref[:, j * _BLK:(j + 1) * _BLK],
                             axis=1, keepdims=True),
                     _NEG)
           for j in range(_NB)]
    bm = jnp.concatenate(bms, axis=1)
    counts = jnp.zeros((_BLK, _NB), jnp.float32)
    for i in range(_NB):
        counts = counts + (bms[i] > bm).astype(jnp.float32)
    keep = counts < float(_TOPK)
    m = jnp.max(jnp.where(keep, bm, _NEG), axis=1, keepdims=True)

    def _pv_chunk(ci):
        parts = []
        for j4 in range(_BPC):
            j = _BPC * ci + j4
            scj = sc_ref[:, j * _BLK:(j + 1) * _BLK]
            parts.append(jnp.where(
                jnp.broadcast_to(keep[:, j:j + 1], (_BLK, _BLK)),
                jnp.exp(scj - m), 0.0))
        ech = jnp.concatenate(parts, axis=1)
        vv = vc_ref[g, pl.ds(ci * _CH, _CH), :]
        return jax.lax.dot_general(ech, vv, (((1,), (0,)), ((), ())),
                                   preferred_element_type=jnp.float32)

    acc_ref[...] = _pv_chunk(0)
    for ci in range(1, _NCH):
        @pl.when(qb >= _BPC * ci)
        def _pv(ci=ci):
            acc_ref[...] = acc_ref[...] + _pv_chunk(ci)

    accv = acc_ref[...]
    ov = accv[:, :_HD] / accv[:, _HD:_HD + 1]
    contrib = jax.lax.dot_general(ov, wo_ref[h], (((1,), (0,)), ((), ())),
                                  preferred_element_type=jnp.float32)

    @pl.when(h == 0)
    def _init():
        y_ref[...] = contrib

    @pl.when(h > 0)
    def _acc():
        y_ref[...] = y_ref[...] + contrib


def kernel(x, freqs_cos, freqs_sin, wq, wk, wv, wo, start_pos):
    b, l, d = x.shape
    cos = jax.lax.dynamic_slice_in_dim(freqs_cos, start_pos, l, axis=0)
    sin = jax.lax.dynamic_slice_in_dim(freqs_sin, start_pos, l, axis=0)

    i = np.arange(_HD)
    src = np.where(i < _HD // 2, 2 * i, 2 * (i - _HD // 2) + 1)
    perm_q = (np.arange(_HQ)[:, None] * _HD + src[None, :]).reshape(-1)
    perm_k = (np.arange(_HKV)[:, None] * _HD + src[None, :]).reshape(-1)
    wq3 = jnp.transpose(wq[perm_q, :].reshape(_HQ, _HD, _D), (0, 2, 1))
    wk3 = jnp.transpose(wk[perm_k, :].reshape(_HKV, _HD, _D), (0, 2, 1))
    wv3 = jnp.transpose(wv.reshape(_HKV, _HD, _D), (0, 2, 1))
    wo3 = jnp.transpose(wo.reshape(_D, _HQ, _HD), (1, 2, 0))
    x2 = x.reshape(l, d)

    y = pl.pallas_call(
        _attn_body,
        grid=(l // _BLK, _HQ),
        in_specs=[
            pl.BlockSpec((_BLK, _D), lambda qb, h: (qb, 0)),
            pl.BlockSpec((_BLK, _HD // 2), lambda qb, h: (qb, 0)),
            pl.BlockSpec((_BLK, _HD // 2), lambda qb, h: (qb, 0)),
            pl.BlockSpec((_HQ, _D, _HD), lambda qb, h: (0, 0, 0)),
            pl.BlockSpec((_HKV, _D, _HD), lambda qb, h: (0, 0, 0)),
            pl.BlockSpec((_HKV, _D, _HD), lambda qb, h: (0, 0, 0)),
            pl.BlockSpec((_HQ, _HD, _D), lambda qb, h: (0, 0, 0)),
        ],
        out_specs=pl.BlockSpec((_BLK, _D), lambda qb, h: (qb, 0)),
        out_shape=jax.ShapeDtypeStruct((l, _D), jnp.float32),
        scratch_shapes=[
            pltpu.VMEM((_HKV, _L, _HD), jnp.float32),
            pltpu.VMEM((_HKV, _L, 2 * _HD), jnp.float32),
            pltpu.VMEM((_BLK, _L), jnp.float32),
            pltpu.VMEM((_BLK, 2 * _HD), jnp.float32),
        ],
        compiler_params=pltpu.CompilerParams(
            dimension_semantics=("arbitrary", "arbitrary")),
    )(x2, cos, sin, wq3, wk3, wv3, wo3)
    return y.reshape(b, l, _D)


# speedup vs baseline: 2.1422x; 2.1422x over previous
import jax
import jax.numpy as jnp
import numpy as np
from jax.experimental import pallas as pl
from jax.experimental.pallas import tpu as pltpu

_L = 2048
_D = 1024
_HQ = 16
_HKV = 4
_HD = 64
_NREP = _HQ // _HKV
_BLK = 128
_NB = _L // _BLK
_TOPK = 8
_MQ = _NREP * _BLK
_SCALE = 1.0 / np.sqrt(_HD)
_NEG = -1e9


def _rope_halves(z, c, s):
    a = z[:, : _HD // 2]
    b = z[:, _HD // 2:]
    return jnp.concatenate([a * c - b * s, a * s + b * c], axis=1)


def _attn_body(x_ref, cos_ref, sin_ref, wq_ref, wk_ref, wv_ref, wo_ref, e_ref,
               y_ref, kc_ref, vc_ref):
    qb = pl.program_id(0)
    g = pl.program_id(1)

    xb = x_ref[...]
    c = cos_ref[...]
    s = sin_ref[...]

    @pl.when(jnp.logical_and(qb == 0, g == 0))
    def _zero_v():
        vc_ref[...] = jnp.zeros((_HKV, _L, 2 * _HD), jnp.float32)

    @pl.when(g == 0)
    def _kv():
        extra = (jax.lax.broadcasted_iota(jnp.int32, (_BLK, _HD), 1) == 0
                 ).astype(jnp.float32)
        for gg in range(_HKV):
            kg = jax.lax.dot_general(xb, wk_ref[gg], (((1,), (0,)), ((), ())),
                                     preferred_element_type=jnp.float32)
            kc_ref[gg, pl.ds(qb * _BLK, _BLK), :] = _rope_halves(kg, c, s)
            vg = jax.lax.dot_general(xb, wv_ref[gg], (((1,), (0,)), ((), ())),
                                     preferred_element_type=jnp.float32)
            vc_ref[gg, pl.ds(qb * _BLK, _BLK), :] = jnp.concatenate(
                [vg, extra], axis=1)

    qp = jax.lax.dot_general(xb, wq_ref[g], (((1,), (0,)), ((), ())),
                             preferred_element_type=jnp.float32)
    qs = jnp.concatenate([qp[:, i * _HD:(i + 1) * _HD] for i in range(_NREP)],
                         axis=0)
    c4 = jnp.concatenate([c] * _NREP, axis=0)
    s4 = jnp.concatenate([s] * _NREP, axis=0)
    qr = _rope_halves(qs, c4, s4)

    kk = kc_ref[g]
    scores = jax.lax.dot_general(qr, kk, (((1,), (1,)), ((), ())),
                                 preferred_element_type=jnp.float32)
    rr = jax.lax.broadcasted_iota(jnp.int32, (_MQ, _L), 0)
    row = qb * _BLK + (rr & (_BLK - 1))
    col = jax.lax.broadcasted_iota(jnp.int32, (_MQ, _L), 1)
    sc = jnp.where(col <= row, scores * _SCALE, _NEG)

    bms = [jnp.max(sc[:, j * _BLK:(j + 1) * _BLK], axis=1, keepdims=True)
           for j in range(_NB)]
    bm = jnp.concatenate(bms, axis=1)
    counts = jnp.zeros((_MQ, _NB), jnp.float32)
    for i in range(_NB):
        counts = counts + (bms[i] > bm).astype(jnp.float32)
    keep = (counts < float(_TOPK)).astype(jnp.float32)
    m = jnp.max(jnp.where(keep > 0.0, bm, _NEG), axis=1, keepdims=True)

    keep_f = jax.lax.dot_general(keep, e_ref[...], (((1,), (0,)), ((), ())),
                                 preferred_element_type=jnp.float32)
    e = jnp.exp(sc - m) * keep_f
    pv = jax.lax.dot_general(e, vc_ref[g], (((1,), (0,)), ((), ())),
                             preferred_element_type=jnp.float32)
    ov = pv[:, :_HD] / pv[:, _HD:_HD + 1]

    ovh = jnp.concatenate([ov[i * _BLK:(i + 1) * _BLK, :] for i in range(_NREP)],
                          axis=1)
    contrib = jax.lax.dot_general(ovh, wo_ref[g], (((1,), (0,)), ((), ())),
                                  preferred_element_type=jnp.float32)

    @pl.when(g == 0)
    def _init():
        y_ref[...] = contrib

    @pl.when(g > 0)
    def _acc():
        y_ref[...] = y_ref[...] + contrib


def kernel(x, freqs_cos, freqs_sin, wq, wk, wv, wo, start_pos):
    b, l, d = x.shape
    cos = jax.lax.dynamic_slice_in_dim(freqs_cos, start_pos, l, axis=0)
    sin = jax.lax.dynamic_slice_in_dim(freqs_sin, start_pos, l, axis=0)

    i = np.arange(_HD)
    src = np.where(i < _HD // 2, 2 * i, 2 * (i - _HD // 2) + 1)
    perm_q = (np.arange(_HQ)[:, None] * _HD + src[None, :]).reshape(-1)
    perm_k = (np.arange(_HKV)[:, None] * _HD + src[None, :]).reshape(-1)
    wq3 = jnp.transpose(wq[perm_q, :].reshape(_HKV, _NREP * _HD, _D), (0, 2, 1))
    wk3 = jnp.transpose(wk[perm_k, :].reshape(_HKV, _HD, _D), (0, 2, 1))
    wv3 = jnp.transpose(wv.reshape(_HKV, _HD, _D), (0, 2, 1))
    wo3 = jnp.transpose(wo.reshape(_D, _HQ, _HD), (1, 2, 0)).reshape(
        _HKV, _NREP * _HD, _D)
    expmat = (np.arange(_L)[None, :] // _BLK ==
              np.arange(_NB)[:, None]).astype(np.float32)
    expmat = jnp.asarray(expmat)
    x2 = x.reshape(l, d)

    y = pl.pallas_call(
        _attn_body,
        grid=(l // _BLK, _HKV),
        in_specs=[
            pl.BlockSpec((_BLK, _D), lambda qb, g: (qb, 0)),
            pl.BlockSpec((_BLK, _HD // 2), lambda qb, g: (qb, 0)),
            pl.BlockSpec((_BLK, _HD // 2), lambda qb, g: (qb, 0)),
            pl.BlockSpec((_HKV, _D, _NREP * _HD), lambda qb, g: (0, 0, 0)),
            pl.BlockSpec((_HKV, _D, _HD), lambda qb, g: (0, 0, 0)),
            pl.BlockSpec((_HKV, _D, _HD), lambda qb, g: (0, 0, 0)),
            pl.BlockSpec((_HKV, _NREP * _HD, _D), lambda qb, g: (0, 0, 0)),
            pl.BlockSpec((_NB, _L), lambda qb, g: (0, 0)),
        ],
        out_specs=pl.BlockSpec((_BLK, _D), lambda qb, g: (qb, 0)),
        out_shape=jax.ShapeDtypeStruct((l, _D), jnp.float32),
        scratch_shapes=[
            pltpu.VMEM((_HKV, _L, _HD), jnp.float32),
            pltpu.VMEM((_HKV, _L, 2 * _HD), jnp.float32),
        ],
        compiler_params=pltpu.CompilerParams(
            dimension_semantics=("arbitrary", "arbitrary")),
    )(x2, cos, sin, wq3, wk3, wv3, wo3, expmat)
    return y.reshape(b, l, _D)
